# Initial kernel scaffold; baseline (speedup 1.0000x reference)
import jax
import jax.numpy as jnp
from jax.experimental import pallas as pl


def kernel(x, W1, b1, W2, b2, W3, b3, W4, b4, fcW, fcb):
    raise NotImplementedError("write your pallas kernel here")


# speedup vs baseline: 3.5390x; 3.5390x over previous
import numpy as np
import jax
import jax.numpy as jnp
from jax.experimental import pallas as pl
from jax.experimental.pallas import tpu as pltpu

_F = 64
_CH = 6
_EDGES = ((1, 0), (2, 0), (0, 1), (2, 1), (1, 2), (3, 2),
          (2, 3), (4, 3), (3, 4), (5, 4), (3, 5), (4, 5))
_MIXROWS = 48
_BLOCK = 6000


def _mix_matrix(fill: float) -> np.ndarray:
    deg = np.zeros((_CH,), np.float64)
    for _, c in _EDGES:
        deg[c] += 1.0
    deg += fill
    dinv = 1.0 / np.sqrt(deg)
    m = np.diag(fill * dinv * dinv)
    for r, c in _EDGES:
        m[c, r] += dinv[r] * dinv[c]
    p = np.eye(_MIXROWS, dtype=np.float64)
    p[:36, :36] = np.kron(m, np.eye(_CH))
    return p.astype(np.float32)


_P1 = jnp.asarray(_mix_matrix(1.0))
_P2 = jnp.asarray(_mix_matrix(2.0))


def _body(x_ref, w1, b1, w2, b2, w3, b3, w4, b4, p1, p2, fcw, fcb, out_ref):
    pid = pl.program_id(0)
    xb = x_ref[...]
    h = xb
    for w, b, p in ((w1, b1, p1), (w2, b2, p1), (w3, b3, p2), (w4, b4, p2)):
        z = jnp.dot(h, w[...], preferred_element_type=jnp.float32)
        top = z[:_MIXROWS]
        mixed = jnp.dot(p[...], top, preferred_element_type=jnp.float32)
        sel = jnp.where(pid == 0, mixed, top)
        z = jnp.concatenate([sel, z[_MIXROWS:]], axis=0)
        h = jnp.maximum(z + b[...], 0.0)
    xr = h + xb
    logits = jnp.sum(xr * fcw[...], axis=1, keepdims=True) + fcb[...]
    out_ref[...] = jax.nn.sigmoid(logits)


def _full(shape):
    return pl.BlockSpec(shape, lambda i: (0,) * len(shape))


@jax.jit
def kernel(x, W1, b1, W2, b2, W3, b3, W4, b4, fcW, fcb):
    n = x.shape[0]
    rows = n * _CH
    xf = x.reshape(rows, _F)
    reps = _BLOCK // _CH
    fcw_t = jnp.tile(fcW, (reps, 1))
    fcb_t = jnp.tile(fcb, reps).reshape(_BLOCK, 1)
    biases = [b.reshape(1, _F) for b in (b1, b2, b3, b4)]

    out = pl.pallas_call(
        _body,
        grid=(rows // _BLOCK,),
        in_specs=[
            pl.BlockSpec((_BLOCK, _F), lambda i: (i, 0)),
            _full((_F, _F)), _full((1, _F)),
            _full((_F, _F)), _full((1, _F)),
            _full((_F, _F)), _full((1, _F)),
            _full((_F, _F)), _full((1, _F)),
            _full((_MIXROWS, _MIXROWS)), _full((_MIXROWS, _MIXROWS)),
            _full((_BLOCK, _F)), _full((_BLOCK, 1)),
        ],
        out_specs=pl.BlockSpec((_BLOCK, 1), lambda i: (i, 0)),
        out_shape=jax.ShapeDtypeStruct((rows, 1), jnp.float32),
        compiler_params=pltpu.CompilerParams(
            dimension_semantics=("arbitrary",),
        ),
    )(xf, W1, biases[0], W2, biases[1], W3, biases[2], W4, biases[3],
      _P1, _P2, fcw_t, fcb_t)
    return out.reshape(n, _CH)


# speedup vs baseline: 13.5068x; 3.8165x over previous
import numpy as np
import jax
import jax.numpy as jnp
from jax.experimental import pallas as pl
from jax.experimental.pallas import tpu as pltpu

_F = 64
_CH = 6
_EDGES = ((1, 0), (2, 0), (0, 1), (2, 1), (1, 2), (3, 2),
          (2, 3), (4, 3), (3, 4), (5, 4), (3, 5), (4, 5))
_MIXLANES = 128
_BLOCK = 2048


def _mix_matrix(fill: float) -> np.ndarray:
    deg = np.zeros((_CH,), np.float64)
    for _, c in _EDGES:
        deg[c] += 1.0
    deg += fill
    dinv = 1.0 / np.sqrt(deg)
    m = np.diag(fill * dinv * dinv)
    for r, c in _EDGES:
        m[c, r] += dinv[r] * dinv[c]
    p = np.eye(_MIXLANES, dtype=np.float64)
    p[:_CH, :_CH] = m.T
    return p.astype(np.float32)


_R1 = _mix_matrix(1.0)
_R2 = _mix_matrix(2.0)


def _body(x_ref, wt1, b1, wt2, b2, wt3, b3, wt4, b4, r1, r2, fcwt, fcb,
          out_ref):
    pid = pl.program_id(0)
    for c in range(_CH):
        xc = x_ref[_F * c:_F * (c + 1), :]
        h = xc
        for wt, b, r in ((wt1, b1, r1), (wt2, b2, r1),
                         (wt3, b3, r2), (wt4, b4, r2)):
            z = jnp.dot(wt[...], h, preferred_element_type=jnp.float32)
            left = z[:, :_MIXLANES]
            mixed = jnp.dot(left, r[...], preferred_element_type=jnp.float32)
            sel = jnp.where(pid == 0, mixed, left)
            z = jnp.concatenate([sel, z[:, _MIXLANES:]], axis=1)
            h = jnp.maximum(z + b[...], 0.0)
        xr = h + xc
        t = xr * fcwt[:, c:c + 1]
        logit = jnp.sum(t, axis=0, keepdims=True) + fcb[0:1, c:c + 1]
        out_ref[c:c + 1, :] = jax.nn.sigmoid(logit)


def _full(shape):
    return pl.BlockSpec(shape, lambda i: (0,) * len(shape))


@jax.jit
def kernel(x, W1, b1, W2, b2, W3, b3, W4, b4, fcW, fcb):
    n = x.shape[0]
    xt = jnp.transpose(x, (1, 2, 0)).reshape(_CH * _F, n)
    wts = [w.T for w in (W1, W2, W3, W4)]
    bcols = [b.reshape(_F, 1) for b in (b1, b2, b3, b4)]

    out = pl.pallas_call(
        _body,
        grid=(pl.cdiv(n, _BLOCK),),
        in_specs=[
            pl.BlockSpec((_CH * _F, _BLOCK), lambda i: (0, i)),
            _full((_F, _F)), _full((_F, 1)),
            _full((_F, _F)), _full((_F, 1)),
            _full((_F, _F)), _full((_F, 1)),
            _full((_F, _F)), _full((_F, 1)),
            _full((_MIXLANES, _MIXLANES)), _full((_MIXLANES, _MIXLANES)),
            _full((_F, _CH)), _full((1, _CH)),
        ],
        out_specs=pl.BlockSpec((_CH, _BLOCK), lambda i: (0, i)),
        out_shape=jax.ShapeDtypeStruct((_CH, n), jnp.float32),
        compiler_params=pltpu.CompilerParams(
            dimension_semantics=("arbitrary",),
        ),
    )(xt, wts[0], bcols[0], wts[1], bcols[1], wts[2], bcols[2],
      wts[3], bcols[3], jnp.asarray(_R1), jnp.asarray(_R2),
      fcW.T, fcb.reshape(1, _CH))
    return out.T
